# baseline (speedup 1.0000x reference)
import functools

import jax
import jax.numpy as jnp
from jax import lax
from jax.experimental import pallas as pl
from jax.experimental.pallas import tpu as pltpu
from jax.experimental.pallas import tpu_sc as plsc

F32 = jnp.float32
D = 128
NC = 2
NS = 16
NW = NC * NS
C = 200
CPAD = 208
NPAD = 5008

_MESH = plsc.VectorSubcoreMesh(core_axis_name="c", subcore_axis_name="s")


def _wid():
    return lax.axis_index("s") * NC + lax.axis_index("c")


def _sc_gather2(fs, fd, si, di):
    E = si.shape[0]
    bpw = E // NW
    nch = bpw // C

    def body(fs_h, fd_h, si_h, di_h, rf_h, rd_h, si_v, di_v, rf_v, rd_v, s1, s2):
        w = _wid()

        def step(i, carry):
            base = w * bpw + i * C
            pltpu.sync_copy(si_h.at[pl.ds(base, C)], si_v)
            pltpu.sync_copy(di_h.at[pl.ds(base, C)], di_v)
            c1 = pltpu.async_copy(fs_h.at[si_v], rf_v, s1)
            c2 = pltpu.async_copy(fd_h.at[di_v], rd_v, s2)
            c1.wait()
            c2.wait()
            pltpu.sync_copy(rf_v, rf_h.at[pl.ds(base, C)])
            pltpu.sync_copy(rd_v, rd_h.at[pl.ds(base, C)])
            return carry

        lax.fori_loop(0, nch, step, 0)

    return pl.kernel(
        body,
        out_type=(jax.ShapeDtypeStruct((E, D), F32),
                  jax.ShapeDtypeStruct((E, D), F32)),
        mesh=_MESH,
        compiler_params=pltpu.CompilerParams(needs_layout_passes=False),
        scratch_types=[
            pltpu.VMEM((C,), jnp.int32),
            pltpu.VMEM((C,), jnp.int32),
            pltpu.VMEM((C, D), F32),
            pltpu.VMEM((C, D), F32),
            pltpu.SemaphoreType.DMA,
            pltpu.SemaphoreType.DMA,
        ],
    )(fs, fd, si, di)


def _sc_gat_layer(fs, fd, si, di, avec, zrows, n):
    E = si.shape[0]
    bpw = E // NW
    nch = bpw // C

    def body(fs_h, fd_h, si_h, di_h, a_h, z_h, p_h, s_h,
             si_v, di_v, rf_v, rd_v, ex_v, a_v, s_loc, shared, sem1, sem2):
        c = lax.axis_index("c")
        s = lax.axis_index("s")
        w = s * NC + c

        def z16(i, carry):
            s_loc[pl.ds(i * 16, 16)] = jnp.zeros((16,), F32)
            return carry

        lax.fori_loop(0, NPAD // 16, z16, 0)
        pltpu.sync_copy(a_h, a_v)

        @pl.when(s == 0)
        def _zero_shared():
            pltpu.sync_copy(z_h, shared)

        plsc.subcore_barrier()

        a_regs = [a_v[pl.ds(kk * 16, 16)] for kk in range(D // 16)]

        def logits16(erow):
            acc = jnp.zeros((16,), F32)
            for k in range(D):
                kv = jnp.full((16,), k, jnp.int32)
                u = plsc.load_gather(rf_v, [erow, kv])
                v = plsc.load_gather(rd_v, [erow, kv])
                t = u + v
                lr = jnp.maximum(t, 0.2 * t)
                acc = acc + a_regs[k // 16][k % 16] * lr
            return jnp.exp(acc)

        def step(i, carry):
            base = w * bpw + i * C
            pltpu.sync_copy(si_h.at[pl.ds(base, C)], si_v)
            pltpu.sync_copy(di_h.at[pl.ds(base, C)], di_v)
            c1 = pltpu.async_copy(fs_h.at[si_v], rf_v.at[pl.ds(0, C)], sem1)
            c2 = pltpu.async_copy(fd_h.at[di_v], rd_v, sem2)
            c1.wait()
            c2.wait()

            def grp_full(g, carry2):
                erow = g * 16 + lax.iota(jnp.int32, 16)
                ex16 = logits16(erow)
                ex_v[pl.ds(g * 16, 16)] = ex16
                return carry2

            lax.fori_loop(0, C // 16, grp_full, 0)
            erow_t = (C - 16) + lax.iota(jnp.int32, 16)
            ex_t = logits16(erow_t)
            mask = lax.iota(jnp.int32, 16) >= 8
            plsc.store_scatter(ex_v, [erow_t], ex_t, mask=mask)

            def g16(gi, cc):
                idx = di_v[pl.ds(gi * 16, 16)]
                val = ex_v[pl.ds(gi * 16, 16)]
                plsc.addupdate_scatter(s_loc, [idx], val)
                return cc

            lax.fori_loop(0, C // 16, g16, 0)
            idx_t = di_v[pl.ds(C - 16, 16)]
            val_t = ex_v[pl.ds(C - 16, 16)]
            plsc.addupdate_scatter(s_loc, [idx_t], val_t, mask=mask)

            def wrow_g(g, cc):
                ex16 = ex_v[pl.ds(g * 16, 16)]
                for j in range(16):
                    e = g * 16 + j
                    exe = ex16[j]
                    for kk in range(D // 16):
                        sl = pl.ds(kk * 16, 16)
                        rf_v[e, sl] = rf_v[e, sl] * exe
                return cc

            lax.fori_loop(0, (C + 15) // 16, wrow_g, 0)
            pltpu.sync_copy(rf_v.at[pl.ds(0, C)], shared.at[di_v], add=True)
            return carry

        lax.fori_loop(0, nch, step, 0)
        plsc.subcore_barrier()

        @pl.when(s == 0)
        def _writeout():
            pltpu.sync_copy(shared, p_h.at[c])

        pltpu.sync_copy(s_loc, s_h.at[w])

    return pl.kernel(
        body,
        out_type=(jax.ShapeDtypeStruct((NC, n, D), F32),
                  jax.ShapeDtypeStruct((NW, NPAD), F32)),
        mesh=_MESH,
        compiler_params=pltpu.CompilerParams(needs_layout_passes=False),
        scratch_types=[
            pltpu.VMEM((C,), jnp.int32),
            pltpu.VMEM((C,), jnp.int32),
            pltpu.VMEM((CPAD, D), F32),
            pltpu.VMEM((C, D), F32),
            pltpu.VMEM((CPAD,), F32),
            pltpu.VMEM((D,), F32),
            pltpu.VMEM((NPAD,), F32),
            pltpu.VMEM_SHARED((n, D), F32),
            pltpu.SemaphoreType.DMA,
            pltpu.SemaphoreType.DMA,
        ],
    )(fs, fd, si, di, avec, zrows)


def _scalar_adds(s_loc, di_v, v_v):
    def g16(gi, carry):
        idx = di_v[pl.ds(gi * 16, 16)]
        val = v_v[pl.ds(gi * 16, 16)]
        plsc.addupdate_scatter(s_loc, [idx], val)
        return carry

    lax.fori_loop(0, C // 16, g16, 0)
    mask = lax.iota(jnp.int32, 16) >= 8
    idx = di_v[pl.ds(C - 16, 16)]
    val = v_v[pl.ds(C - 16, 16)]
    plsc.addupdate_scatter(s_loc, [idx], val, mask=mask)


def _sc_scatter(wrows, ex, di, zrows, n):
    E = di.shape[0]
    bpw = E // NW
    nch = bpw // C

    def body(w_h, ex_h, di_h, z_h, p_h, s_h, di_v, r_v, ex_v, s_loc, shared, sem):
        c = lax.axis_index("c")
        s = lax.axis_index("s")
        w = s * NC + c

        def z16(i, carry):
            s_loc[pl.ds(i * 16, 16)] = jnp.zeros((16,), F32)
            return carry

        lax.fori_loop(0, NPAD // 16, z16, 0)

        @pl.when(s == 0)
        def _zero_shared():
            pltpu.sync_copy(z_h, shared)

        plsc.subcore_barrier()

        def step(i, carry):
            base = w * bpw + i * C
            pltpu.sync_copy(di_h.at[pl.ds(base, C)], di_v)
            pltpu.sync_copy(w_h.at[pl.ds(base, C)], r_v)
            pltpu.sync_copy(ex_h.at[pl.ds(base, C)], ex_v)
            pltpu.sync_copy(r_v, shared.at[di_v], add=True)
            _scalar_adds(s_loc, di_v, ex_v)
            return carry

        lax.fori_loop(0, nch, step, 0)
        plsc.subcore_barrier()

        @pl.when(s == 0)
        def _writeout():
            pltpu.sync_copy(shared, p_h.at[c])

        pltpu.sync_copy(s_loc, s_h.at[w])

    return pl.kernel(
        body,
        out_type=(jax.ShapeDtypeStruct((NC, n, D), F32),
                  jax.ShapeDtypeStruct((NW, NPAD), F32)),
        mesh=_MESH,
        compiler_params=pltpu.CompilerParams(needs_layout_passes=False),
        scratch_types=[
            pltpu.VMEM((C,), jnp.int32),
            pltpu.VMEM((C, D), F32),
            pltpu.VMEM((C,), F32),
            pltpu.VMEM((NPAD,), F32),
            pltpu.VMEM_SHARED((n, D), F32),
            pltpu.SemaphoreType.DMA,
        ],
    )(wrows, ex, di, zrows)


def _sc_spmm(x, si, di, zrows, n):
    E = si.shape[0]
    bpw = E // NW
    nch = bpw // C

    def body(x_h, si_h, di_h, z_h, a_h, si_v, di_v, r_v, shared, sem):
        c = lax.axis_index("c")
        s = lax.axis_index("s")
        w = s * NC + c

        @pl.when(s == 0)
        def _zero_shared():
            pltpu.sync_copy(z_h, shared)

        plsc.subcore_barrier()

        def step(i, carry):
            base = w * bpw + i * C
            pltpu.sync_copy(si_h.at[pl.ds(base, C)], si_v)
            pltpu.sync_copy(di_h.at[pl.ds(base, C)], di_v)
            pltpu.async_copy(x_h.at[si_v], r_v, sem).wait()
            pltpu.sync_copy(r_v, shared.at[di_v], add=True)
            return carry

        lax.fori_loop(0, nch, step, 0)
        plsc.subcore_barrier()

        @pl.when(s == 0)
        def _writeout():
            pltpu.sync_copy(shared, a_h.at[c])

    return pl.kernel(
        body,
        out_type=jax.ShapeDtypeStruct((NC, n, D), F32),
        mesh=_MESH,
        compiler_params=pltpu.CompilerParams(needs_layout_passes=False),
        scratch_types=[
            pltpu.VMEM((C,), jnp.int32),
            pltpu.VMEM((C,), jnp.int32),
            pltpu.VMEM((C, D), F32),
            pltpu.VMEM_SHARED((n, D), F32),
            pltpu.SemaphoreType.DMA,
        ],
    )(x, si, di, zrows)


def _sc_degree(di):
    E = di.shape[0]
    bpw = E // NW
    nch = bpw // C

    def body(di_h, s_h, di_v, s_loc):
        w = _wid()

        def z16(i, carry):
            s_loc[pl.ds(i * 16, 16)] = jnp.zeros((16,), F32)
            return carry

        lax.fori_loop(0, NPAD // 16, z16, 0)

        def step(i, carry):
            base = w * bpw + i * C
            pltpu.sync_copy(di_h.at[pl.ds(base, C)], di_v)

            def g16(gi, cc):
                idx = di_v[pl.ds(gi * 16, 16)]
                plsc.addupdate_scatter(s_loc, [idx], jnp.ones((16,), F32))
                return cc

            lax.fori_loop(0, C // 16, g16, 0)
            mask = lax.iota(jnp.int32, 16) >= 8
            idx = di_v[pl.ds(C - 16, 16)]
            plsc.addupdate_scatter(s_loc, [idx], jnp.ones((16,), F32),
                                   mask=mask)
            return carry

        lax.fori_loop(0, nch, step, 0)
        pltpu.sync_copy(s_loc, s_h.at[w])

    return pl.kernel(
        body,
        out_type=jax.ShapeDtypeStruct((NW, NPAD), F32),
        mesh=_MESH,
        compiler_params=pltpu.CompilerParams(needs_layout_passes=False),
        scratch_types=[
            pltpu.VMEM((C,), jnp.int32),
            pltpu.VMEM((NPAD,), F32),
        ],
    )(di)


def _dot(a, b):
    return jnp.dot(a, b, preferred_element_type=F32)


def _tc_mm6(U, I, w1, w2, w3, w4, w5, w6):
    def body(u, i_, a, b, c, d, e, f, o1, o2, o3, o4, o5, o6):
        uu = u[...]
        ii = i_[...]
        o1[...] = _dot(uu, a[...])
        o2[...] = _dot(ii, b[...])
        o3[...] = _dot(ii, c[...])
        o4[...] = _dot(uu, d[...])
        o5[...] = _dot(uu, e[...])
        o6[...] = _dot(uu, f[...])

    n = U.shape[0]
    sh = jax.ShapeDtypeStruct((n, D), F32)
    return pl.pallas_call(body, out_shape=(sh,) * 6)(U, I, w1, w2, w3, w4, w5, w6)


def _tc_edge(rf, rd, a):
    E = rf.shape[0]
    BE = 6400
    grid = E // BE

    def body(rf_ref, rd_ref, a_ref, ex_ref, w_ref):
        f = rf_ref[...]
        t = f + rd_ref[...]
        l = jnp.where(t >= 0, t, 0.2 * t)
        ex = jnp.exp(jnp.sum(l * a_ref[...], axis=1, keepdims=True))
        ex_ref[...] = ex
        w_ref[...] = f * ex

    return pl.pallas_call(
        body,
        grid=(grid,),
        in_specs=[
            pl.BlockSpec((BE, D), lambda i: (i, 0)),
            pl.BlockSpec((BE, D), lambda i: (i, 0)),
            pl.BlockSpec((1, D), lambda i: (0, 0)),
        ],
        out_specs=[
            pl.BlockSpec((BE, 1), lambda i: (i, 0)),
            pl.BlockSpec((BE, D), lambda i: (i, 0)),
        ],
        out_shape=[
            jax.ShapeDtypeStruct((E, 1), F32),
            jax.ShapeDtypeStruct((E, D), F32),
        ],
    )(rf, rd, a.reshape(1, D))


def _fin(p_ref, s_ref, b_ref, n):
    ssum = jnp.sum(s_ref[...], axis=0)[:n]
    return (p_ref[0] + p_ref[1]) / (ssum + 1e-9)[:, None] + b_ref[...]


def _tc_fin_mm(P, S, bprev, w):
    n = P.shape[1]

    def body(p, s, b, w_ref, o):
        o[...] = _dot(_fin(p, s, b, n), w_ref[...])

    return pl.pallas_call(
        body, out_shape=jax.ShapeDtypeStruct((n, D), F32),
    )(P, S, bprev.reshape(1, D), w)


def _tc_fin2_mm(P3, S3, b3, P4, S4, b4, wa, wb, bout):
    n = P3.shape[1]

    def body(p3, s3, b3r, p4, s4, b4r, wa_r, wb_r, bo, o):
        h3 = _fin(p3, s3, b3r, n)
        h4 = _fin(p4, s4, b4r, n)
        o[...] = _dot(h3, wa_r[...]) + _dot(h4, wb_r[...]) + bo[...]

    return pl.pallas_call(
        body, out_shape=jax.ShapeDtypeStruct((n, D), F32),
    )(P3, S3, b3.reshape(1, D), P4, S4, b4.reshape(1, D), wa, wb,
      bout.reshape(1, D))


def _dinv_of(s_ref, n):
    deg = jnp.sum(s_ref[...], axis=0)[:n]
    return jnp.where(deg > 0, lax.rsqrt(jnp.maximum(deg, 1.0)), 0.0)


def _tc_xn(x, Sdeg):
    n = x.shape[0]

    def body(x_ref, s_ref, o):
        o[...] = x_ref[...] * _dinv_of(s_ref, n)[:, None]

    return pl.pallas_call(body, out_shape=jax.ShapeDtypeStruct((n, D), F32))(
        x, Sdeg)


def _tc_t1(T0, A0, Sdeg, lam):
    n = T0.shape[0]

    def body(t0, a0, s_ref, lam_ref, o_t1, o_xn1):
        dinv = _dinv_of(s_ref, n)[:, None]
        t0v = t0[...]
        lap0 = t0v - (a0[0] + a0[1]) * dinv
        re = 2.0 / lam_ref[0, 0]
        t1 = re * lap0 - t0v
        o_t1[...] = t1
        o_xn1[...] = t1 * dinv

    sh = jax.ShapeDtypeStruct((n, D), F32)
    return pl.pallas_call(body, out_shape=(sh, sh))(
        T0, A0, Sdeg, lam.reshape(1, 1))


def _tc_cheb_sp(T0, T1, A1, Sdeg, lam, w_cheb, b_cheb, ws_sp, wd_sp):
    n = T0.shape[0]

    def body(t0, t1, a1, s_ref, lam_ref, wc, bc, ws, wd, o_fs, o_fd):
        dinv = _dinv_of(s_ref, n)[:, None]
        t0v = t0[...]
        t1v = t1[...]
        lap1 = t1v - (a1[0] + a1[1]) * dinv
        re = 2.0 / lam_ref[0, 0]
        t2 = 2.0 * re * lap1 - 2.0 * t1v - t0v
        h = (_dot(t0v, wc[0]) + _dot(t1v, wc[1]) + _dot(t2, wc[2]) + bc[...])
        o_fs[...] = _dot(h, ws[...])
        o_fd[...] = _dot(h, wd[...])

    sh = jax.ShapeDtypeStruct((n, D), F32)
    return pl.pallas_call(body, out_shape=(sh, sh))(
        T0, T1, A1, Sdeg, lam.reshape(1, 1), w_cheb, b_cheb.reshape(1, D),
        ws_sp, wd_sp)


def _tc_mutual(P5, S5, b_sp, user_pref, U,
               wc_a, wc_b, b_cons, wsoc_a, wsoc_b, b_soc,
               wmp_a, wmp_b, b_mp, wms_a, wms_b, b_ms):
    n = U.shape[0]

    def body(p5, s5, bsp, up, u, wca, wcb, bc, wsa, wsb, bs,
             wpa, wpb, bp, wma, wmb, bm, o_p, o_s):
        us = _fin(p5, s5, bsp, n)
        uu = u[...]
        h_uP = _dot(up[...], wca[...]) + _dot(uu, wcb[...]) + bc[...]
        h_uS = _dot(us, wsa[...]) + _dot(uu, wsb[...]) + bs[...]
        h_m = h_uP * h_uS
        h_mP = h_m * jax.nn.softmax(h_uP, axis=1)
        h_mS = h_m * jax.nn.softmax(h_uS, axis=1)
        o_p[...] = _dot(h_mP, wpa[...]) + _dot(h_uP, wpb[...]) + bp[...]
        o_s[...] = _dot(h_mS, wma[...]) + _dot(h_uS, wmb[...]) + bm[...]

    sh = jax.ShapeDtypeStruct((n, D), F32)
    return pl.pallas_call(body, out_shape=(sh, sh))(
        P5, S5, b_sp.reshape(1, D), user_pref, U,
        wc_a, wc_b, b_cons.reshape(1, D), wsoc_a, wsoc_b, b_soc.reshape(1, D),
        wmp_a, wmp_b, b_mp.reshape(1, D), wms_a, wms_b, b_ms.reshape(1, D))


def _tc_score(x, y):
    n, m = x.shape[0], y.shape[0]
    BN = 1000

    def body(x_ref, y_ref, o):
        o[...] = lax.dot_general(
            x_ref[...], y_ref[...], (((1,), (1,)), ((), ())),
            preferred_element_type=F32)

    return pl.pallas_call(
        body,
        grid=(n // BN,),
        in_specs=[
            pl.BlockSpec((BN, D), lambda i: (i, 0)),
            pl.BlockSpec((m, D), lambda i: (0, 0)),
        ],
        out_specs=pl.BlockSpec((BN, m), lambda i: (i, 0)),
        out_shape=jax.ShapeDtypeStruct((n, m), F32),
    )(x, y)


def kernel(user_table, item_table, rate_edge_index, friend_edge_index,
           laplacian_lambda_max,
           w_src_g1r, w_dst_g1r, a_g1r, b_g1r,
           w_src_g1d, w_dst_g1d, a_g1d, b_g1d,
           w_src_g2d, w_dst_g2d, a_g2d, b_g2d,
           w_src_g2f, w_dst_g2f, a_g2f, b_g2f,
           w_src_sp, w_dst_sp, a_sp, b_sp,
           w_out, b_out, w_cheb, b_cheb,
           w_cons, b_cons, w_soc, b_soc,
           w_mp, b_mp, w_ms, b_ms):
    U = user_table
    I = item_table
    Nu = U.shape[0]
    Ni = I.shape[0]
    u_src = rate_edge_index[0]
    i_dst = rate_edge_index[1]
    f_src = friend_edge_index[0]
    f_dst = friend_edge_index[1]
    zu = jnp.zeros((Nu, D), F32)
    zi = jnp.zeros((Ni, D), F32)

    fs1, fd1, fs2, fd2, fd3, fd4 = _tc_mm6(
        U, I, w_src_g1r, w_dst_g1r, w_src_g1d, w_dst_g1d, w_dst_g2d,
        w_dst_g2f)

    def gat_layer(fs, fd, si, di, a, zrows, n):
        return _sc_gat_layer(fs, fd, si, di, a, zrows, n)

    P1, S1 = gat_layer(fs1, fd1, u_src, i_dst, a_g1r, zi, Ni)
    P2, S2 = gat_layer(fs2, fd2, i_dst, u_src, a_g1d, zu, Nu)
    fs3 = _tc_fin_mm(P1, S1, b_g1r, w_src_g2d)
    P3, S3 = gat_layer(fs3, fd3, i_dst, u_src, a_g2d, zu, Nu)
    fs4 = _tc_fin_mm(P2, S2, b_g1d, w_src_g2f)
    P4, S4 = gat_layer(fs4, fd4, f_src, f_dst, a_g2f, zu, Nu)
    user_pref = _tc_fin2_mm(P3, S3, b_g2d, P4, S4, b_g2f,
                            w_out[:D], w_out[D:], b_out)

    Sdeg = _sc_degree(f_dst)
    xn0 = _tc_xn(U, Sdeg)
    A0 = _sc_spmm(xn0, f_src, f_dst, zu, Nu)
    T1, xn1 = _tc_t1(U, A0, Sdeg, laplacian_lambda_max)
    A1 = _sc_spmm(xn1, f_src, f_dst, zu, Nu)
    fs5, fd5 = _tc_cheb_sp(U, T1, A1, Sdeg, laplacian_lambda_max,
                           w_cheb, b_cheb, w_src_sp, w_dst_sp)
    P5, S5 = gat_layer(fs5, fd5, f_src, f_dst, a_sp, zu, Nu)

    h_new_P, h_new_S = _tc_mutual(
        P5, S5, b_sp, user_pref, U,
        w_cons[:D], w_cons[D:], b_cons, w_soc[:D], w_soc[D:], b_soc,
        w_mp[:D], w_mp[D:], b_mp, w_ms[:D], w_ms[D:], b_ms)

    r_hat = _tc_score(h_new_P, I)
    s_hat = _tc_score(h_new_S, U)
    return (r_hat, s_hat)


# speedup vs baseline: 3.0385x; 3.0385x over previous
import functools

import jax
import jax.numpy as jnp
from jax import lax
from jax.experimental import pallas as pl
from jax.experimental.pallas import tpu as pltpu
from jax.experimental.pallas import tpu_sc as plsc

F32 = jnp.float32
D = 128
NC = 2
NS = 16
NW = NC * NS
C = 200
NPAD = 5008

_MESH = plsc.VectorSubcoreMesh(core_axis_name="c", subcore_axis_name="s")


def _wid():
    return lax.axis_index("s") * NC + lax.axis_index("c")


def _sc_gather2(fs, fd, si, di):
    E = si.shape[0]
    bpw = E // NW
    nch = bpw // C
    si3 = si.reshape(NW, nch, C)
    di3 = di.reshape(NW, nch, C)

    def body(fs_h, fd_h, si_h, di_h, rf_h, rd_h, si_a, di_a,
             siv0, siv1, div0, div1, rf0, rf1, rd0, rd1, sg0, sg1, sw0, sw1):
        w = _wid()
        rf_s = (rf0, rf1)
        rd_s = (rd0, rd1)
        siv = (siv0, siv1)
        div = (div0, div1)
        sg = (sg0, sg1)
        sw = (sw0, sw1)
        pltpu.sync_copy(si_h.at[w], si_a)
        pltpu.sync_copy(di_h.at[w], di_a)

        def start_g(i, b):
            _row_copy(si_a, i, siv[b])
            _row_copy(di_a, i, div[b])
            pltpu.async_copy(fs_h.at[siv[b]], rf_s[b], sg[b])
            pltpu.async_copy(fd_h.at[div[b]], rd_s[b], sg[b])

        def wait_g(i, b):
            pltpu.make_async_copy(fs_h.at[siv[b]], rf_s[b], sg[b]).wait()
            pltpu.make_async_copy(fd_h.at[div[b]], rd_s[b], sg[b]).wait()

        def start_w(i, b):
            base = w * bpw + i * C
            pltpu.async_copy(rf_s[b], rf_h.at[pl.ds(base, C)], sw[b])
            pltpu.async_copy(rd_s[b], rd_h.at[pl.ds(base, C)], sw[b])

        def wait_w(i, b):
            base = w * bpw + i * C
            pltpu.make_async_copy(rf_s[b], rf_h.at[pl.ds(base, C)], sw[b]).wait()
            pltpu.make_async_copy(rd_s[b], rd_h.at[pl.ds(base, C)], sw[b]).wait()

        start_g(0, 0)

        def pair(p, carry):
            i = 2 * p
            wait_g(i, 0)
            start_w(i, 0)
            @pl.when(p >= 1)
            def _():
                wait_w(i - 1, 1)
            start_g(i + 1, 1)
            wait_g(i + 1, 1)
            start_w(i + 1, 1)
            wait_w(i, 0)
            @pl.when(i + 2 < nch)
            def _():
                start_g(i + 2, 0)
            return carry

        lax.fori_loop(0, nch // 2, pair, 0)
        i = nch - 1
        wait_g(i, 0)
        start_w(i, 0)
        wait_w(i - 1, 1)
        wait_w(i, 0)

    return pl.kernel(
        body,
        out_type=(jax.ShapeDtypeStruct((E, D), F32),
                  jax.ShapeDtypeStruct((E, D), F32)),
        mesh=_MESH,
        compiler_params=pltpu.CompilerParams(needs_layout_passes=False),
        scratch_types=[
            pltpu.VMEM((nch, C), jnp.int32),
            pltpu.VMEM((nch, C), jnp.int32),
            pltpu.VMEM((C,), jnp.int32),
            pltpu.VMEM((C,), jnp.int32),
            pltpu.VMEM((C,), jnp.int32),
            pltpu.VMEM((C,), jnp.int32),
            pltpu.VMEM((C, D), F32),
            pltpu.VMEM((C, D), F32),
            pltpu.VMEM((C, D), F32),
            pltpu.VMEM((C, D), F32),
            pltpu.SemaphoreType.DMA,
            pltpu.SemaphoreType.DMA,
            pltpu.SemaphoreType.DMA,
            pltpu.SemaphoreType.DMA,
        ],
    )(fs, fd, si3, di3)


def _row_copy(src2d, i, dst1d):
    def cp(g, cc):
        dst1d[pl.ds(g * 16, 16)] = src2d[i, pl.ds(g * 16, 16)]
        return cc

    lax.fori_loop(0, C // 16, cp, 0)
    lanes = lax.iota(jnp.int32, 16)
    vals = src2d[i, pl.ds(C - 16, 16)]
    plsc.store_scatter(dst1d, [(C - 16) + lanes], vals, mask=lanes >= 8)


def _scalar_adds(s_loc, di_v, v_v):
    def g16(gi, carry):
        idx = di_v[pl.ds(gi * 16, 16)]
        val = v_v[pl.ds(gi * 16, 16)]
        plsc.addupdate_scatter(s_loc, [idx], val)
        return carry

    lax.fori_loop(0, C // 16, g16, 0)
    mask = lax.iota(jnp.int32, 16) >= 8
    idx = di_v[pl.ds(C - 16, 16)]
    val = v_v[pl.ds(C - 16, 16)]
    plsc.addupdate_scatter(s_loc, [idx], val, mask=mask)


def _sc_scatter(wrows, ex, di, zrows, n):
    E = di.shape[0]
    bpw = E // NW
    nch = bpw // C

    di3 = di.reshape(NW, nch, C)
    ex3 = ex.reshape(NW, nch, C)

    def body(w_h, ex_h, di_h, z_h, p_h, s_h, di_a, ex_a, di_v, r0, r1,
             s_loc, shared, sl0, sl1):
        c = lax.axis_index("c")
        s = lax.axis_index("s")
        w = s * NC + c
        r_s = (r0, r1)
        sl = (sl0, sl1)

        def z16(i, carry):
            s_loc[pl.ds(i * 16, 16)] = jnp.zeros((16,), F32)
            return carry

        lax.fori_loop(0, NPAD // 16, z16, 0)
        pltpu.sync_copy(di_h.at[w], di_a)
        pltpu.sync_copy(ex_h.at[w], ex_a)

        @pl.when(s == 0)
        def _zero_shared():
            pltpu.sync_copy(z_h, shared)

        plsc.subcore_barrier()

        def start_l(i, b):
            base = w * bpw + i * C
            pltpu.async_copy(w_h.at[pl.ds(base, C)], r_s[b], sl[b])

        def wait_l(i, b):
            base = w * bpw + i * C
            pltpu.make_async_copy(w_h.at[pl.ds(base, C)], r_s[b], sl[b]).wait()

        def chunk(i, b):
            _row_copy(di_a, i, di_v)
            wait_l(i, b)
            pltpu.sync_copy(r_s[b], shared.at[di_v], add=True)

            def g16(gi, cc):
                idx = di_a[i, pl.ds(gi * 16, 16)]
                val = ex_a[i, pl.ds(gi * 16, 16)]
                plsc.addupdate_scatter(s_loc, [idx], val)
                return cc

            lax.fori_loop(0, C // 16, g16, 0)
            mask = lax.iota(jnp.int32, 16) >= 8
            idx = di_a[i, pl.ds(C - 16, 16)]
            val = ex_a[i, pl.ds(C - 16, 16)]
            plsc.addupdate_scatter(s_loc, [idx], val, mask=mask)

        start_l(0, 0)

        def pair(p, carry):
            i = 2 * p
            start_l(i + 1, 1)
            chunk(i, 0)
            @pl.when(i + 2 < nch)
            def _():
                start_l(i + 2, 0)
            chunk(i + 1, 1)
            return carry

        lax.fori_loop(0, nch // 2, pair, 0)
        chunk(nch - 1, 0)
        plsc.subcore_barrier()

        @pl.when(s == 0)
        def _writeout():
            pltpu.sync_copy(shared, p_h.at[c])

        pltpu.sync_copy(s_loc, s_h.at[w])

    return pl.kernel(
        body,
        out_type=(jax.ShapeDtypeStruct((NC, n, D), F32),
                  jax.ShapeDtypeStruct((NW, NPAD), F32)),
        mesh=_MESH,
        compiler_params=pltpu.CompilerParams(needs_layout_passes=False),
        scratch_types=[
            pltpu.VMEM((nch, C), jnp.int32),
            pltpu.VMEM((nch, C), F32),
            pltpu.VMEM((C,), jnp.int32),
            pltpu.VMEM((C, D), F32),
            pltpu.VMEM((C, D), F32),
            pltpu.VMEM((NPAD,), F32),
            pltpu.VMEM_SHARED((n, D), F32),
            pltpu.SemaphoreType.DMA,
            pltpu.SemaphoreType.DMA,
        ],
    )(wrows, ex3, di3, zrows)


def _sc_spmm(x, si, di, zrows, n):
    E = si.shape[0]
    bpw = E // NW
    nch = bpw // C

    si3 = si.reshape(NW, nch, C)
    di3 = di.reshape(NW, nch, C)

    def body(x_h, si_h, di_h, z_h, a_h, si_a, di_a, siv0, siv1, di_v,
             r0, r1, shared, sg0, sg1):
        c = lax.axis_index("c")
        s = lax.axis_index("s")
        w = s * NC + c
        r_s = (r0, r1)
        siv = (siv0, siv1)
        sg = (sg0, sg1)
        pltpu.sync_copy(si_h.at[w], si_a)
        pltpu.sync_copy(di_h.at[w], di_a)

        @pl.when(s == 0)
        def _zero_shared():
            pltpu.sync_copy(z_h, shared)

        plsc.subcore_barrier()

        def start_g(i, b):
            _row_copy(si_a, i, siv[b])
            pltpu.async_copy(x_h.at[siv[b]], r_s[b], sg[b])

        def wait_g(i, b):
            pltpu.make_async_copy(x_h.at[siv[b]], r_s[b], sg[b]).wait()

        def chunk(i, b):
            _row_copy(di_a, i, di_v)
            wait_g(i, b)
            pltpu.sync_copy(r_s[b], shared.at[di_v], add=True)

        start_g(0, 0)

        def pair(p, carry):
            i = 2 * p
            start_g(i + 1, 1)
            chunk(i, 0)
            @pl.when(i + 2 < nch)
            def _():
                start_g(i + 2, 0)
            chunk(i + 1, 1)
            return carry

        lax.fori_loop(0, nch // 2, pair, 0)
        chunk(nch - 1, 0)
        plsc.subcore_barrier()

        @pl.when(s == 0)
        def _writeout():
            pltpu.sync_copy(shared, a_h.at[c])

    return pl.kernel(
        body,
        out_type=jax.ShapeDtypeStruct((NC, n, D), F32),
        mesh=_MESH,
        compiler_params=pltpu.CompilerParams(needs_layout_passes=False),
        scratch_types=[
            pltpu.VMEM((nch, C), jnp.int32),
            pltpu.VMEM((nch, C), jnp.int32),
            pltpu.VMEM((C,), jnp.int32),
            pltpu.VMEM((C,), jnp.int32),
            pltpu.VMEM((C,), jnp.int32),
            pltpu.VMEM((C, D), F32),
            pltpu.VMEM((C, D), F32),
            pltpu.VMEM_SHARED((n, D), F32),
            pltpu.SemaphoreType.DMA,
            pltpu.SemaphoreType.DMA,
        ],
    )(x, si3, di3, zrows)


def _sc_degree(di):
    E = di.shape[0]
    bpw = E // NW
    nch = bpw // C

    di3 = di.reshape(NW, nch, C)

    def body(di_h, s_h, di_a, s_loc):
        w = _wid()

        def z16(i, carry):
            s_loc[pl.ds(i * 16, 16)] = jnp.zeros((16,), F32)
            return carry

        lax.fori_loop(0, NPAD // 16, z16, 0)
        pltpu.sync_copy(di_h.at[w], di_a)

        def step(i, carry):
            def g16(gi, cc):
                idx = di_a[i, pl.ds(gi * 16, 16)]
                plsc.addupdate_scatter(s_loc, [idx], jnp.ones((16,), F32))
                return cc

            lax.fori_loop(0, C // 16, g16, 0)
            mask = lax.iota(jnp.int32, 16) >= 8
            idx = di_a[i, pl.ds(C - 16, 16)]
            plsc.addupdate_scatter(s_loc, [idx], jnp.ones((16,), F32),
                                   mask=mask)
            return carry

        lax.fori_loop(0, nch, step, 0)
        pltpu.sync_copy(s_loc, s_h.at[w])

    return pl.kernel(
        body,
        out_type=jax.ShapeDtypeStruct((NW, NPAD), F32),
        mesh=_MESH,
        compiler_params=pltpu.CompilerParams(needs_layout_passes=False),
        scratch_types=[
            pltpu.VMEM((nch, C), jnp.int32),
            pltpu.VMEM((NPAD,), F32),
        ],
    )(di3)


def _dot(a, b):
    return jnp.dot(a, b, preferred_element_type=F32)


def _tc_mm6(U, I, w1, w2, w3, w4, w5, w6):
    def body(u, i_, a, b, c, d, e, f, o1, o2, o3, o4, o5, o6):
        uu = u[...]
        ii = i_[...]
        o1[...] = _dot(uu, a[...])
        o2[...] = _dot(ii, b[...])
        o3[...] = _dot(ii, c[...])
        o4[...] = _dot(uu, d[...])
        o5[...] = _dot(uu, e[...])
        o6[...] = _dot(uu, f[...])

    n = U.shape[0]
    sh = jax.ShapeDtypeStruct((n, D), F32)
    return pl.pallas_call(body, out_shape=(sh,) * 6)(U, I, w1, w2, w3, w4, w5, w6)


def _tc_edge(rf, rd, a):
    E = rf.shape[0]
    BE = 6400
    grid = E // BE

    def body(rf_ref, rd_ref, a_ref, ex_ref, w_ref):
        f = rf_ref[...]
        t = f + rd_ref[...]
        l = jnp.where(t >= 0, t, 0.2 * t)
        ex = jnp.exp(jnp.sum(l * a_ref[...], axis=1, keepdims=True))
        ex_ref[...] = ex
        w_ref[...] = f * ex

    return pl.pallas_call(
        body,
        grid=(grid,),
        in_specs=[
            pl.BlockSpec((BE, D), lambda i: (i, 0)),
            pl.BlockSpec((BE, D), lambda i: (i, 0)),
            pl.BlockSpec((1, D), lambda i: (0, 0)),
        ],
        out_specs=[
            pl.BlockSpec((BE, 1), lambda i: (i, 0)),
            pl.BlockSpec((BE, D), lambda i: (i, 0)),
        ],
        out_shape=[
            jax.ShapeDtypeStruct((E, 1), F32),
            jax.ShapeDtypeStruct((E, D), F32),
        ],
    )(rf, rd, a.reshape(1, D))


def _fin(p_ref, s_ref, b_ref, n):
    ssum = jnp.sum(s_ref[...], axis=0)[:n]
    return (p_ref[0] + p_ref[1]) / (ssum + 1e-9)[:, None] + b_ref[...]


def _tc_fin_mm(P, S, bprev, w):
    n = P.shape[1]

    def body(p, s, b, w_ref, o):
        o[...] = _dot(_fin(p, s, b, n), w_ref[...])

    return pl.pallas_call(
        body, out_shape=jax.ShapeDtypeStruct((n, D), F32),
    )(P, S, bprev.reshape(1, D), w)


def _tc_fin2_mm(P3, S3, b3, P4, S4, b4, wa, wb, bout):
    n = P3.shape[1]

    def body(p3, s3, b3r, p4, s4, b4r, wa_r, wb_r, bo, o):
        h3 = _fin(p3, s3, b3r, n)
        h4 = _fin(p4, s4, b4r, n)
        o[...] = _dot(h3, wa_r[...]) + _dot(h4, wb_r[...]) + bo[...]

    return pl.pallas_call(
        body, out_shape=jax.ShapeDtypeStruct((n, D), F32),
    )(P3, S3, b3.reshape(1, D), P4, S4, b4.reshape(1, D), wa, wb,
      bout.reshape(1, D))


def _dinv_of(s_ref, n):
    deg = jnp.sum(s_ref[...], axis=0)[:n]
    return jnp.where(deg > 0, lax.rsqrt(jnp.maximum(deg, 1.0)), 0.0)


def _tc_xn(x, Sdeg):
    n = x.shape[0]

    def body(x_ref, s_ref, o):
        o[...] = x_ref[...] * _dinv_of(s_ref, n)[:, None]

    return pl.pallas_call(body, out_shape=jax.ShapeDtypeStruct((n, D), F32))(
        x, Sdeg)


def _tc_t1(T0, A0, Sdeg, lam):
    n = T0.shape[0]

    def body(t0, a0, s_ref, lam_ref, o_t1, o_xn1):
        dinv = _dinv_of(s_ref, n)[:, None]
        t0v = t0[...]
        lap0 = t0v - (a0[0] + a0[1]) * dinv
        re = 2.0 / lam_ref[0, 0]
        t1 = re * lap0 - t0v
        o_t1[...] = t1
        o_xn1[...] = t1 * dinv

    sh = jax.ShapeDtypeStruct((n, D), F32)
    return pl.pallas_call(body, out_shape=(sh, sh))(
        T0, A0, Sdeg, lam.reshape(1, 1))


def _tc_cheb_sp(T0, T1, A1, Sdeg, lam, w_cheb, b_cheb, ws_sp, wd_sp):
    n = T0.shape[0]

    def body(t0, t1, a1, s_ref, lam_ref, wc, bc, ws, wd, o_fs, o_fd):
        dinv = _dinv_of(s_ref, n)[:, None]
        t0v = t0[...]
        t1v = t1[...]
        lap1 = t1v - (a1[0] + a1[1]) * dinv
        re = 2.0 / lam_ref[0, 0]
        t2 = 2.0 * re * lap1 - 2.0 * t1v - t0v
        h = (_dot(t0v, wc[0]) + _dot(t1v, wc[1]) + _dot(t2, wc[2]) + bc[...])
        o_fs[...] = _dot(h, ws[...])
        o_fd[...] = _dot(h, wd[...])

    sh = jax.ShapeDtypeStruct((n, D), F32)
    return pl.pallas_call(body, out_shape=(sh, sh))(
        T0, T1, A1, Sdeg, lam.reshape(1, 1), w_cheb, b_cheb.reshape(1, D),
        ws_sp, wd_sp)


def _tc_mutual(P5, S5, b_sp, user_pref, U,
               wc_a, wc_b, b_cons, wsoc_a, wsoc_b, b_soc,
               wmp_a, wmp_b, b_mp, wms_a, wms_b, b_ms):
    n = U.shape[0]

    def body(p5, s5, bsp, up, u, wca, wcb, bc, wsa, wsb, bs,
             wpa, wpb, bp, wma, wmb, bm, o_p, o_s):
        us = _fin(p5, s5, bsp, n)
        uu = u[...]
        h_uP = _dot(up[...], wca[...]) + _dot(uu, wcb[...]) + bc[...]
        h_uS = _dot(us, wsa[...]) + _dot(uu, wsb[...]) + bs[...]
        h_m = h_uP * h_uS
        h_mP = h_m * jax.nn.softmax(h_uP, axis=1)
        h_mS = h_m * jax.nn.softmax(h_uS, axis=1)
        o_p[...] = _dot(h_mP, wpa[...]) + _dot(h_uP, wpb[...]) + bp[...]
        o_s[...] = _dot(h_mS, wma[...]) + _dot(h_uS, wmb[...]) + bm[...]

    sh = jax.ShapeDtypeStruct((n, D), F32)
    return pl.pallas_call(body, out_shape=(sh, sh))(
        P5, S5, b_sp.reshape(1, D), user_pref, U,
        wc_a, wc_b, b_cons.reshape(1, D), wsoc_a, wsoc_b, b_soc.reshape(1, D),
        wmp_a, wmp_b, b_mp.reshape(1, D), wms_a, wms_b, b_ms.reshape(1, D))


def _tc_score(x, y):
    n, m = x.shape[0], y.shape[0]
    BN = 1000

    def body(x_ref, y_ref, o):
        o[...] = lax.dot_general(
            x_ref[...], y_ref[...], (((1,), (1,)), ((), ())),
            preferred_element_type=F32)

    return pl.pallas_call(
        body,
        grid=(n // BN,),
        in_specs=[
            pl.BlockSpec((BN, D), lambda i: (i, 0)),
            pl.BlockSpec((m, D), lambda i: (0, 0)),
        ],
        out_specs=pl.BlockSpec((BN, m), lambda i: (i, 0)),
        out_shape=jax.ShapeDtypeStruct((n, m), F32),
    )(x, y)


def kernel(user_table, item_table, rate_edge_index, friend_edge_index,
           laplacian_lambda_max,
           w_src_g1r, w_dst_g1r, a_g1r, b_g1r,
           w_src_g1d, w_dst_g1d, a_g1d, b_g1d,
           w_src_g2d, w_dst_g2d, a_g2d, b_g2d,
           w_src_g2f, w_dst_g2f, a_g2f, b_g2f,
           w_src_sp, w_dst_sp, a_sp, b_sp,
           w_out, b_out, w_cheb, b_cheb,
           w_cons, b_cons, w_soc, b_soc,
           w_mp, b_mp, w_ms, b_ms):
    U = user_table
    I = item_table
    Nu = U.shape[0]
    Ni = I.shape[0]
    u_src = rate_edge_index[0]
    i_dst = rate_edge_index[1]
    f_src = friend_edge_index[0]
    f_dst = friend_edge_index[1]
    zu = jnp.zeros((Nu, D), F32)
    zi = jnp.zeros((Ni, D), F32)

    fs1, fd1, fs2, fd2, fd3, fd4 = _tc_mm6(
        U, I, w_src_g1r, w_dst_g1r, w_src_g1d, w_dst_g1d, w_dst_g2d,
        w_dst_g2f)

    def gat_layer(fs, fd, si, di, a, zrows, n):
        rf, rd = _sc_gather2(fs, fd, si, di)
        ex, wrows = _tc_edge(rf, rd, a)
        return _sc_scatter(wrows, ex.reshape(-1), di, zrows, n)

    P1, S1 = gat_layer(fs1, fd1, u_src, i_dst, a_g1r, zi, Ni)
    P2, S2 = gat_layer(fs2, fd2, i_dst, u_src, a_g1d, zu, Nu)
    fs3 = _tc_fin_mm(P1, S1, b_g1r, w_src_g2d)
    P3, S3 = gat_layer(fs3, fd3, i_dst, u_src, a_g2d, zu, Nu)
    fs4 = _tc_fin_mm(P2, S2, b_g1d, w_src_g2f)
    P4, S4 = gat_layer(fs4, fd4, f_src, f_dst, a_g2f, zu, Nu)
    user_pref = _tc_fin2_mm(P3, S3, b_g2d, P4, S4, b_g2f,
                            w_out[:D], w_out[D:], b_out)

    Sdeg = _sc_degree(f_dst)
    xn0 = _tc_xn(U, Sdeg)
    A0 = _sc_spmm(xn0, f_src, f_dst, zu, Nu)
    T1, xn1 = _tc_t1(U, A0, Sdeg, laplacian_lambda_max)
    A1 = _sc_spmm(xn1, f_src, f_dst, zu, Nu)
    fs5, fd5 = _tc_cheb_sp(U, T1, A1, Sdeg, laplacian_lambda_max,
                           w_cheb, b_cheb, w_src_sp, w_dst_sp)
    P5, S5 = gat_layer(fs5, fd5, f_src, f_dst, a_sp, zu, Nu)

    h_new_P, h_new_S = _tc_mutual(
        P5, S5, b_sp, user_pref, U,
        w_cons[:D], w_cons[D:], b_cons, w_soc[:D], w_soc[D:], b_soc,
        w_mp[:D], w_mp[D:], b_mp, w_ms[:D], w_ms[D:], b_ms)

    r_hat = _tc_score(h_new_P, I)
    s_hat = _tc_score(h_new_S, U)
    return (r_hat, s_hat)
